# baseline (speedup 1.0000x reference)
import jax
import jax.numpy as jnp
from jax import lax
from jax.experimental import pallas as pl
from jax.experimental.pallas import tpu as pltpu
from jax.experimental.pallas import tpu_sc as plsc

NC = 2
NS = 16
L = 16
NW = NC * NS
B = 16384
D = 64
BPW = B // NW
CH = 32
NCH = BPW // CH


def _mf_body(user_hbm, item_hbm, uw_hbm, iw_hbm, out_hbm,
             uidx_v, iidx_v, du_v, di_v, out_v, sem, sem2, sem3, sem4):
    sems = (sem, sem2, sem3, sem4)
    wid = lax.axis_index("s") * NC + lax.axis_index("c")
    base = wid * BPW
    pltpu.sync_copy(user_hbm.at[pl.ds(base, BPW)], uidx_v)
    pltpu.sync_copy(item_hbm.at[pl.ds(base, BPW)], iidx_v)
    lane = lax.iota(jnp.int32, L)

    def chunk_body(c, carry):
        for g in range(CH // L):
            uvec = uidx_v[pl.ds(c * CH + g * L, L)]
            ivec = iidx_v[pl.ds(c * CH + g * L, L)]
            for j in range(L):
                r_u = uvec[j]
                r_i = ivec[j]
                pltpu.async_copy(uw_hbm.at[r_u >> 3], du_v.at[g * L + j],
                                 sems[j % 4])
                pltpu.async_copy(iw_hbm.at[r_i >> 3], di_v.at[g * L + j],
                                 sems[(j + 1) % 4])
        for j in range(2 * CH):
            pltpu.make_async_copy(uw_hbm.at[0], du_v.at[0], sems[j % 4]).wait()
        for g in range(CH // L):
            uvec = uidx_v[pl.ds(c * CH + g * L, L)]
            ivec = iidx_v[pl.ds(c * CH + g * L, L)]
            su = lax.bitwise_and(uvec, 7)
            si = lax.bitwise_and(ivec, 7)
            bvec = g * L + lane

            def t_body(t, acc):
                col = lax.bitwise_and(lane + t, D - 1)
                a = plsc.load_gather(du_v, [bvec, su, col])
                b = plsc.load_gather(di_v, [bvec, si, col])
                return acc + a * b

            acc = lax.fori_loop(0, D, t_body, jnp.zeros((L,), jnp.float32))
            out_v[pl.ds(c * CH + g * L, L)] = acc
        return carry

    lax.fori_loop(0, NCH, chunk_body, 0)
    pltpu.sync_copy(out_v, out_hbm.at[pl.ds(base, BPW)])


def kernel(user, item, user_emb_w, item_emb_w):
    mesh = plsc.VectorSubcoreMesh(core_axis_name="c", subcore_axis_name="s")
    f = pl.kernel(
        _mf_body,
        out_type=jax.ShapeDtypeStruct((B,), jnp.float32),
        mesh=mesh,
        scratch_types=[
            pltpu.VMEM((BPW,), jnp.int32),
            pltpu.VMEM((BPW,), jnp.int32),
            pltpu.VMEM((CH, 8, D), jnp.float32),
            pltpu.VMEM((CH, 8, D), jnp.float32),
            pltpu.VMEM((BPW,), jnp.float32),
            pltpu.SemaphoreType.DMA,
            pltpu.SemaphoreType.DMA,
            pltpu.SemaphoreType.DMA,
            pltpu.SemaphoreType.DMA,
        ],
        compiler_params=pltpu.CompilerParams(needs_layout_passes=False),
    )
    nq = user_emb_w.shape[0] // 8
    return f(user.astype(jnp.int32), item.astype(jnp.int32),
             user_emb_w.reshape(nq, 8, D), item_emb_w.reshape(nq, 8, D))


# speedup vs baseline: 1.0505x; 1.0505x over previous
import jax
import jax.numpy as jnp
from jax import lax
from jax.experimental import pallas as pl
from jax.experimental.pallas import tpu as pltpu
from jax.experimental.pallas import tpu_sc as plsc

NC = 2
NS = 16
L = 16
NW = NC * NS
B = 16384
D = 64
BPW = B // NW
CH = 32
NCH = BPW // CH


def _mf_body(user_hbm, item_hbm, uw_hbm, iw_hbm, out_hbm,
             uidx_v, iidx_v, du_v, di_v, out_v, sem, sem2, sem3, sem4):
    sems = (sem, sem2, sem3, sem4)
    wid = lax.axis_index("s") * NC + lax.axis_index("c")
    base = wid * BPW
    pltpu.sync_copy(user_hbm.at[pl.ds(base, BPW)], uidx_v)
    pltpu.sync_copy(item_hbm.at[pl.ds(base, BPW)], iidx_v)
    lane = lax.iota(jnp.int32, L)

    def chunk_body(c, carry):
        for g in range(CH // L):
            uvec = uidx_v[pl.ds(c * CH + g * L, L)]
            ivec = iidx_v[pl.ds(c * CH + g * L, L)]
            for j in range(L):
                r_u = uvec[j]
                r_i = ivec[j]
                pltpu.async_copy(
                    uw_hbm.at[r_u >> 3, pl.ds(((r_u >> 2) & 1) * 4, 4)],
                    du_v.at[g * L + j], sems[j % 4])
                pltpu.async_copy(
                    iw_hbm.at[r_i >> 3, pl.ds(((r_i >> 2) & 1) * 4, 4)],
                    di_v.at[g * L + j], sems[(j + 1) % 4])
        for j in range(2 * CH):
            pltpu.make_async_copy(uw_hbm.at[0, pl.ds(0, 4)], du_v.at[0],
                                  sems[j % 4]).wait()
        for g in range(CH // L):
            uvec = uidx_v[pl.ds(c * CH + g * L, L)]
            ivec = iidx_v[pl.ds(c * CH + g * L, L)]
            su = lax.bitwise_and(uvec, 3)
            si = lax.bitwise_and(ivec, 3)
            bvec = g * L + lane

            def t_body(t, acc):
                col = lax.bitwise_and(lane + t, D - 1)
                a = plsc.load_gather(du_v, [bvec, su, col])
                b = plsc.load_gather(di_v, [bvec, si, col])
                return acc + a * b

            acc = lax.fori_loop(0, D, t_body, jnp.zeros((L,), jnp.float32))
            out_v[pl.ds(c * CH + g * L, L)] = acc
        return carry

    lax.fori_loop(0, NCH, chunk_body, 0)
    pltpu.sync_copy(out_v, out_hbm.at[pl.ds(base, BPW)])


def kernel(user, item, user_emb_w, item_emb_w):
    mesh = plsc.VectorSubcoreMesh(core_axis_name="c", subcore_axis_name="s")
    f = pl.kernel(
        _mf_body,
        out_type=jax.ShapeDtypeStruct((B,), jnp.float32),
        mesh=mesh,
        scratch_types=[
            pltpu.VMEM((BPW,), jnp.int32),
            pltpu.VMEM((BPW,), jnp.int32),
            pltpu.VMEM((CH, 4, D), jnp.float32),
            pltpu.VMEM((CH, 4, D), jnp.float32),
            pltpu.VMEM((BPW,), jnp.float32),
            pltpu.SemaphoreType.DMA,
            pltpu.SemaphoreType.DMA,
            pltpu.SemaphoreType.DMA,
            pltpu.SemaphoreType.DMA,
        ],
        compiler_params=pltpu.CompilerParams(needs_layout_passes=False),
    )
    nq = user_emb_w.shape[0] // 8
    return f(user.astype(jnp.int32), item.astype(jnp.int32),
             user_emb_w.reshape(nq, 8, D), item_emb_w.reshape(nq, 8, D))


# speedup vs baseline: 1.0899x; 1.0375x over previous
import jax
import jax.numpy as jnp
from jax import lax
from jax.experimental import pallas as pl
from jax.experimental.pallas import tpu as pltpu
from jax.experimental.pallas import tpu_sc as plsc

NC = 2
NS = 16
L = 16
NW = NC * NS
B = 16384
D = 64
BPW = B // NW
CH = 32
NCH = BPW // CH


def _mf_body(user_hbm, item_hbm, uw_hbm, iw_hbm, out_hbm,
             uidx_v, iidx_v, du_v, di_v, out_v, sem, sem2, sem3, sem4):
    sems = (sem, sem2, sem3, sem4)
    wid = lax.axis_index("s") * NC + lax.axis_index("c")
    base = wid * BPW
    pltpu.sync_copy(user_hbm.at[pl.ds(base, BPW)], uidx_v)
    pltpu.sync_copy(item_hbm.at[pl.ds(base, BPW)], iidx_v)
    lane = lax.iota(jnp.int32, L)

    def chunk_body(c, carry):
        for g in range(CH // L):
            uvec = uidx_v[pl.ds(c * CH + g * L, L)]
            ivec = iidx_v[pl.ds(c * CH + g * L, L)]
            for j in range(L):
                r_u = uvec[j]
                r_i = ivec[j]
                pltpu.async_copy(
                    uw_hbm.at[r_u >> 3, pl.ds(r_u & 7, 1)],
                    du_v.at[g * L + j], sems[j % 4])
                pltpu.async_copy(
                    iw_hbm.at[r_i >> 3, pl.ds(r_i & 7, 1)],
                    di_v.at[g * L + j], sems[(j + 1) % 4])
        for j in range(2 * CH):
            pltpu.make_async_copy(uw_hbm.at[0, pl.ds(0, 1)], du_v.at[0],
                                  sems[j % 4]).wait()
        for g in range(CH // L):
            uvec = uidx_v[pl.ds(c * CH + g * L, L)]
            ivec = iidx_v[pl.ds(c * CH + g * L, L)]
            su = jnp.zeros((L,), jnp.int32)
            si = jnp.zeros((L,), jnp.int32)
            bvec = g * L + lane

            def t_body(t, acc):
                col = lax.bitwise_and(lane + t, D - 1)
                a = plsc.load_gather(du_v, [bvec, su, col])
                b = plsc.load_gather(di_v, [bvec, si, col])
                return acc + a * b

            acc = lax.fori_loop(0, D, t_body, jnp.zeros((L,), jnp.float32))
            out_v[pl.ds(c * CH + g * L, L)] = acc
        return carry

    lax.fori_loop(0, NCH, chunk_body, 0)
    pltpu.sync_copy(out_v, out_hbm.at[pl.ds(base, BPW)])


def kernel(user, item, user_emb_w, item_emb_w):
    mesh = plsc.VectorSubcoreMesh(core_axis_name="c", subcore_axis_name="s")
    f = pl.kernel(
        _mf_body,
        out_type=jax.ShapeDtypeStruct((B,), jnp.float32),
        mesh=mesh,
        scratch_types=[
            pltpu.VMEM((BPW,), jnp.int32),
            pltpu.VMEM((BPW,), jnp.int32),
            pltpu.VMEM((CH, 1, D), jnp.float32),
            pltpu.VMEM((CH, 1, D), jnp.float32),
            pltpu.VMEM((BPW,), jnp.float32),
            pltpu.SemaphoreType.DMA,
            pltpu.SemaphoreType.DMA,
            pltpu.SemaphoreType.DMA,
            pltpu.SemaphoreType.DMA,
        ],
        compiler_params=pltpu.CompilerParams(needs_layout_passes=False),
    )
    nq = user_emb_w.shape[0] // 8
    return f(user.astype(jnp.int32), item.astype(jnp.int32),
             user_emb_w.reshape(nq, 8, D), item_emb_w.reshape(nq, 8, D))


# speedup vs baseline: 1.1058x; 1.0146x over previous
import jax
import jax.numpy as jnp
from jax import lax
from jax.experimental import pallas as pl
from jax.experimental.pallas import tpu as pltpu
from jax.experimental.pallas import tpu_sc as plsc

NC = 2
NS = 16
L = 16
NW = NC * NS
B = 16384
D = 64
BPW = B // NW
NG = BPW // L


def _mf_body(user_hbm, item_hbm, uw_hbm, iw_hbm, out_hbm,
             uidx_v, iidx_v, du_v, di_v, out_v, sem):
    wid = lax.axis_index("s") * NC + lax.axis_index("c")
    base = wid * BPW
    pltpu.sync_copy(user_hbm.at[pl.ds(base, BPW)], uidx_v)
    pltpu.sync_copy(item_hbm.at[pl.ds(base, BPW)], iidx_v)
    lane = lax.iota(jnp.int32, L)

    NGH = NG // 2
    for h in range(2):
        hb = h * NGH * L

        def issue_body(g, carry):
            uvec = uidx_v[pl.ds(hb + g * L, L)]
            ivec = iidx_v[pl.ds(hb + g * L, L)]
            for j in range(L):
                r_u = uvec[j]
                r_i = ivec[j]
                pltpu.async_copy(
                    uw_hbm.at[r_u >> 3, pl.ds(r_u & 7, 1)],
                    du_v.at[g * L + j], sem)
                pltpu.async_copy(
                    iw_hbm.at[r_i >> 3, pl.ds(r_i & 7, 1)],
                    di_v.at[g * L + j], sem)
            return carry

        lax.fori_loop(0, NGH, issue_body, 0)

        def comp_body(g, carry):
            for j in range(2 * L):
                pltpu.make_async_copy(uw_hbm.at[0, pl.ds(0, 1)], du_v.at[0],
                                      sem).wait()
            bvec = g * L + lane
            zero = jnp.zeros((L,), jnp.int32)

            def t_body(t, acc):
                col = lax.bitwise_and(lane + t, D - 1)
                a = plsc.load_gather(du_v, [bvec, zero, col])
                b = plsc.load_gather(di_v, [bvec, zero, col])
                return acc + a * b

            acc = lax.fori_loop(0, D, t_body, jnp.zeros((L,), jnp.float32))
            out_v[pl.ds(hb + g * L, L)] = acc
            return carry

        lax.fori_loop(0, NGH, comp_body, 0)
    pltpu.sync_copy(out_v, out_hbm.at[pl.ds(base, BPW)])


def kernel(user, item, user_emb_w, item_emb_w):
    mesh = plsc.VectorSubcoreMesh(core_axis_name="c", subcore_axis_name="s")
    f = pl.kernel(
        _mf_body,
        out_type=jax.ShapeDtypeStruct((B,), jnp.float32),
        mesh=mesh,
        scratch_types=[
            pltpu.VMEM((BPW,), jnp.int32),
            pltpu.VMEM((BPW,), jnp.int32),
            pltpu.VMEM((BPW // 2, 1, D), jnp.float32),
            pltpu.VMEM((BPW // 2, 1, D), jnp.float32),
            pltpu.VMEM((BPW,), jnp.float32),
            pltpu.SemaphoreType.DMA,
        ],
        compiler_params=pltpu.CompilerParams(needs_layout_passes=False),
    )
    nq = user_emb_w.shape[0] // 8
    return f(user.astype(jnp.int32), item.astype(jnp.int32),
             user_emb_w.reshape(nq, 8, D), item_emb_w.reshape(nq, 8, D))
